# speedup vs baseline: 2.3565x; 2.3565x over previous
import jax
import jax.numpy as jnp
from jax.experimental import pallas as pl
from jax.experimental.pallas import tpu as pltpu


def _ln_body(x_ref, pos_ref, gamma_ref, beta_ref, out_ref):
    e = x_ref[0] + pos_ref[...]
    mean = jnp.mean(e, axis=-1, keepdims=True)
    d = e - mean
    var = jnp.mean(d * d, axis=-1, keepdims=True)
    inv = jax.lax.rsqrt(var + 1e-6)
    out_ref[0] = d * inv * gamma_ref[...] + beta_ref[...]


def kernel(input_ids, pos_emb, gamma, beta):
    B, S, H = input_ids.shape
    BS = 512
    grid = (S // BS, B)
    gamma2 = gamma.reshape(1, H)
    beta2 = beta.reshape(1, H)
    return pl.pallas_call(
        _ln_body,
        grid=grid,
        in_specs=[
            pl.BlockSpec((1, BS, H), lambda s, b: (b, s, 0)),
            pl.BlockSpec((BS, H), lambda s, b: (s, 0)),
            pl.BlockSpec((1, H), lambda s, b: (0, 0)),
            pl.BlockSpec((1, H), lambda s, b: (0, 0)),
        ],
        out_specs=pl.BlockSpec((1, BS, H), lambda s, b: (b, s, 0)),
        out_shape=jax.ShapeDtypeStruct((B, S, H), jnp.float32),
        compiler_params=pltpu.CompilerParams(
            dimension_semantics=("arbitrary", "arbitrary")
        ),
    )(input_ids, pos_emb, gamma2, beta2)


# speedup vs baseline: 2.7487x; 1.1664x over previous
import jax
import jax.numpy as jnp
from jax.experimental import pallas as pl
from jax.experimental.pallas import tpu as pltpu


def _ln_body(x_ref, pos_ref, gamma_ref, beta_ref, out_ref):
    e = x_ref[0] + pos_ref[...]
    mean = jnp.mean(e, axis=-1, keepdims=True)
    d = e - mean
    var = jnp.mean(d * d, axis=-1, keepdims=True)
    inv = jax.lax.rsqrt(var + 1e-6)
    out_ref[0] = d * inv * gamma_ref[...] + beta_ref[...]


def kernel(input_ids, pos_emb, gamma, beta):
    B, S, H = input_ids.shape
    BS = 1024
    grid = (S // BS, B)
    gamma2 = gamma.reshape(1, H)
    beta2 = beta.reshape(1, H)
    return pl.pallas_call(
        _ln_body,
        grid=grid,
        in_specs=[
            pl.BlockSpec((1, BS, H), lambda s, b: (b, s, 0)),
            pl.BlockSpec((BS, H), lambda s, b: (s, 0)),
            pl.BlockSpec((1, H), lambda s, b: (0, 0)),
            pl.BlockSpec((1, H), lambda s, b: (0, 0)),
        ],
        out_specs=pl.BlockSpec((1, BS, H), lambda s, b: (b, s, 0)),
        out_shape=jax.ShapeDtypeStruct((B, S, H), jnp.float32),
        compiler_params=pltpu.CompilerParams(
            dimension_semantics=("arbitrary", "arbitrary")
        ),
    )(input_ids, pos_emb, gamma2, beta2)


# speedup vs baseline: 2.7694x; 1.0075x over previous
import jax
import jax.numpy as jnp
from jax.experimental import pallas as pl
from jax.experimental.pallas import tpu as pltpu


def _ln_body(x_ref, pos_ref, gamma_ref, beta_ref, out_ref):
    e = x_ref[0] + pos_ref[...]
    mean = jnp.mean(e, axis=-1, keepdims=True)
    d = e - mean
    var = jnp.mean(d * d, axis=-1, keepdims=True)
    inv = jax.lax.rsqrt(var + 1e-6)
    out_ref[0] = d * inv * gamma_ref[...] + beta_ref[...]


def kernel(input_ids, pos_emb, gamma, beta):
    B, S, H = input_ids.shape
    BS = 2048
    grid = (S // BS, B)
    gamma2 = gamma.reshape(1, H)
    beta2 = beta.reshape(1, H)
    return pl.pallas_call(
        _ln_body,
        grid=grid,
        in_specs=[
            pl.BlockSpec((1, BS, H), lambda s, b: (b, s, 0)),
            pl.BlockSpec((BS, H), lambda s, b: (s, 0)),
            pl.BlockSpec((1, H), lambda s, b: (0, 0)),
            pl.BlockSpec((1, H), lambda s, b: (0, 0)),
        ],
        out_specs=pl.BlockSpec((1, BS, H), lambda s, b: (b, s, 0)),
        out_shape=jax.ShapeDtypeStruct((B, S, H), jnp.float32),
        compiler_params=pltpu.CompilerParams(
            dimension_semantics=("arbitrary", "arbitrary")
        ),
    )(input_ids, pos_emb, gamma2, beta2)
